# Initial kernel scaffold; baseline (speedup 1.0000x reference)
import jax
import jax.numpy as jnp
from jax.experimental import pallas as pl


def kernel(l0, l1, l2, cb0, cb1, cb2):
    raise NotImplementedError("write your pallas kernel here")


# speedup vs baseline: 3.7264x; 3.7264x over previous
import functools

import jax
import jax.numpy as jnp
from jax.experimental import pallas as pl
from jax.experimental.pallas import tpu as pltpu

_D = 384
_COSTS = (0.05, 0.25, 0.6)


def _vq_body(x_ref, cb_ref, idx_ref, q_ref, loss_ref, *, n_codes):
    x = x_ref[...]
    cb = cb_ref[...]
    x2 = jnp.sum(x * x, axis=1, keepdims=True)
    cb2 = jnp.sum(cb * cb, axis=1)[None, :]
    xc = jax.lax.dot_general(x, cb, (((1,), (1,)), ((), ())),
                             preferred_element_type=jnp.float32)
    d2 = x2 - 2.0 * xc + cb2
    m = jnp.min(d2, axis=1, keepdims=True)
    iota = jax.lax.broadcasted_iota(jnp.int32, d2.shape, 1)
    idx = jnp.min(jnp.where(d2 == m, iota, n_codes), axis=1)
    idx_ref[...] = idx
    onehot = (iota == idx[:, None]).astype(jnp.float32)
    q = jax.lax.dot_general(onehot, cb, (((1,), (0,)), ((), ())),
                            preferred_element_type=jnp.float32)
    q_ref[...] = q
    e = q - x
    s = jnp.sum(e * e)

    @pl.when(pl.program_id(0) == 0)
    def _init():
        loss_ref[0, 0] = 0.0

    loss_ref[0, 0] += s


def _vq_level(x_flat, cb, block_rows):
    n, d = x_flat.shape
    k = cb.shape[0]
    grid = n // block_rows
    body = functools.partial(_vq_body, n_codes=k)
    idx, q, loss_sum = pl.pallas_call(
        body,
        grid=(grid,),
        in_specs=[
            pl.BlockSpec((block_rows, d), lambda i: (i, 0)),
            pl.BlockSpec((k, d), lambda i: (0, 0)),
        ],
        out_specs=[
            pl.BlockSpec((block_rows,), lambda i: (i,)),
            pl.BlockSpec((block_rows, d), lambda i: (i, 0)),
            pl.BlockSpec((1, 1), lambda i: (0, 0), memory_space=pltpu.SMEM),
        ],
        out_shape=[
            jax.ShapeDtypeStruct((n,), jnp.int32),
            jax.ShapeDtypeStruct((n, d), jnp.float32),
            jax.ShapeDtypeStruct((1, 1), jnp.float32),
        ],
    )(x_flat, cb)
    return idx, q, loss_sum[0, 0]


def kernel(l0, l1, l2, cb0, cb1, cb2):
    levels = ((l0, cb0, 1024), (l1, cb1, 1024), (l2, cb2, 1024))
    idxs, qs, sums = [], [], []
    for x, cb, br in levels:
        xf = x.reshape(-1, _D)
        idx, q, s = _vq_level(xf, cb, br)
        idxs.append(idx.reshape(x.shape[:-1]))
        qs.append(q.reshape(x.shape))
        sums.append(s)
    total = (
        (1.0 + _COSTS[0]) * sums[0] / l0.size
        + (1.0 + _COSTS[1]) * sums[1] / l1.size
        + (1.0 + _COSTS[2]) * sums[2] / l2.size
    )
    return (idxs[0], idxs[1], idxs[2], total, qs[0], qs[1], qs[2])
